# Initial kernel scaffold; baseline (speedup 1.0000x reference)
import jax
import jax.numpy as jnp
from jax.experimental import pallas as pl


def kernel(x, dim):
    raise NotImplementedError("write your pallas kernel here")


# speedup vs baseline: 3.3889x; 3.3889x over previous
import functools

import jax
import jax.numpy as jnp
from jax import lax
from jax.experimental import pallas as pl
from jax.experimental.pallas import tpu as pltpu
from jax.experimental.pallas import tpu_sc as plsc

_NC = 2
_NS = 16
_L = 16
_HB = 256
_MIN32 = -2147483648


def _monokey(xv):
    b = lax.bitcast_convert_type(xv, jnp.int32)
    sgn = jnp.right_shift(b, 31)
    mag = jnp.bitwise_and(b, jnp.int32(0x7FFFFFFF))
    return jnp.bitwise_xor(mag, sgn) - sgn


def _digit(key, shift):
    u = jnp.bitwise_xor(key, _MIN32)
    sh = jnp.full(key.shape, shift, jnp.int32)
    return jnp.bitwise_and(lax.shift_right_logical(u, sh), jnp.int32(0xFF))


@functools.lru_cache(maxsize=None)
def _build(nrows, ncols, k, interpret=False):
    nv = ncols // _L
    nw = _NC * _NS
    rpw = nrows // nw
    mesh = plsc.VectorSubcoreMesh(
        core_axis_name="c", subcore_axis_name="s",
        num_cores=_NC, num_subcores=_NS)

    def body(x_hbm, out_hbm, xrow, cand, hist, suffix, outv):
        c = lax.axis_index("c")
        s = lax.axis_index("s")
        wid = s * _NC + c
        lane = lax.iota(jnp.int32, _L)
        ones = jnp.ones((_L,), jnp.int32)
        zeros = jnp.zeros((_L,), jnp.int32)

        def clear_hist():
            def clr(i, carry):
                hist[pl.ds(i * _L, _L)] = zeros
                return carry
            lax.fori_loop(0, _L * _HB // _L, clr, 0)

        def hist_pass_x():
            clear_hist()

            def hb(i, carry):
                xv = xrow[pl.ds(i * _L, _L)]
                d = _digit(_monokey(xv), 24)
                plsc.addupdate_scatter(hist, [lane * _HB + d], ones)
                return carry
            lax.fori_loop(0, nv, hb, 0)

        def hist_pass_cand(n, shift):
            clear_hist()
            nvreg = (n + _L - 1) // _L

            def hb(i, carry):
                valid = (i * _L + lane) < n
                key = cand[pl.ds(i * _L, _L)]
                d = _digit(key, shift)
                plsc.addupdate_scatter(hist, [lane * _HB + d], ones,
                                       mask=valid)
                return carry
            lax.fori_loop(0, nvreg, hb, 0)

        def find_bucket(n, kk):
            for step in (8, 4, 2, 1):
                def red(i, carry):
                    a = hist[pl.ds(i * _L, _L)]
                    b2 = hist[pl.ds(step * _HB + i * _L, _L)]
                    hist[pl.ds(i * _L, _L)] = a + b2
                    return carry
                lax.fori_loop(0, step * _HB // _L, red, 0)
            suffix[pl.ds(_HB, _L)] = zeros
            carry = jnp.int32(0)
            bcount = zeros
            for jj in range(_HB // _L):
                t = hist[pl.ds(jj * _L, _L)]
                csum = plsc.cumsum(t)
                excl = (csum - t) + carry
                suf = n - excl
                suffix[pl.ds(jj * _L, _L)] = suf
                carry = carry + jnp.sum(t)
                bcount = bcount + plsc.all_reduce_population_count(suf >= kk)
            above_v = plsc.load_gather(suffix, [bcount])
            at_v = plsc.load_gather(suffix, [bcount - 1])
            above = jnp.max(above_v)
            bsel = jnp.max(bcount) - 1
            return bsel, kk - above, jnp.max(at_v) - above

        def compact_x(b0):
            def cb(i, p):
                xv = xrow[pl.ds(i * _L, _L)]
                key = _monokey(xv)
                m = _digit(key, 24) == b0
                mi = m.astype(jnp.int32)
                r = plsc.cumsum(mi) - mi
                plsc.store_scatter(cand, [p + r], key, mask=m)
                return p + plsc.all_reduce_population_count(m)
            lax.fori_loop(0, nv, cb, zeros)

        def compact_cand(n, b, shift):
            nvreg = (n + _L - 1) // _L

            def cb(i, p):
                valid = (i * _L + lane) < n
                key = cand[pl.ds(i * _L, _L)]
                m = valid & (_digit(key, shift) == b)
                mi = m.astype(jnp.int32)
                r = plsc.cumsum(mi) - mi
                plsc.store_scatter(cand, [p + r], key, mask=m)
                return p + plsc.all_reduce_population_count(m)
            lax.fori_loop(0, nvreg, cb, zeros)

        def final_pass(tkey, kk):
            def fb(i, pe):
                p, eqs = pe
                xv = xrow[pl.ds(i * _L, _L)]
                key = _monokey(xv)
                mgt = key > tkey
                meq = key == tkey
                ceq = plsc.cumsum(meq.astype(jnp.int32))
                m = mgt | (meq & ((eqs + ceq) <= kk))
                mi = m.astype(jnp.int32)
                r = plsc.cumsum(mi) - mi
                plsc.store_scatter(outv, [p + r], xv, mask=m)
                return (p + plsc.all_reduce_population_count(m),
                        eqs + plsc.all_reduce_population_count(meq))
            lax.fori_loop(0, nv, fb, (zeros, zeros))

        def per_row(j, carry):
            row = wid * rpw + j
            pltpu.sync_copy(x_hbm.at[row], xrow)
            hist_pass_x()
            b0, kk, n_cur = find_bucket(jnp.int32(ncols), jnp.int32(k))
            compact_x(b0)
            tu = jnp.left_shift(b0, 24)
            for lvl in (1, 2, 3):
                shift = 24 - 8 * lvl
                hist_pass_cand(n_cur, shift)
                b, kk, n_next = find_bucket(n_cur, kk)
                tu = jnp.bitwise_or(tu, jnp.left_shift(b, shift))
                if lvl < 3:
                    compact_cand(n_cur, b, shift)
                n_cur = n_next
            tkey = jnp.bitwise_xor(tu, _MIN32)
            final_pass(tkey, kk)
            pltpu.sync_copy(outv.at[pl.ds(0, k)], out_hbm.at[row])
            return carry
        lax.fori_loop(0, rpw, per_row, 0)

    return pl.kernel(
        body,
        out_type=jax.ShapeDtypeStruct((nrows, k), jnp.float32),
        mesh=mesh,
        scratch_types=[
            pltpu.VMEM((ncols,), jnp.float32),
            pltpu.VMEM((ncols + _L,), jnp.int32),
            pltpu.VMEM((_L * _HB,), jnp.int32),
            pltpu.VMEM((_HB + _L,), jnp.int32),
            pltpu.VMEM((k + _L,), jnp.float32),
        ],
        compiler_params=pltpu.CompilerParams(needs_layout_passes=False),
        interpret=interpret,
    )


def kernel(x, dim):
    del dim
    nrows, ncols = x.shape
    return _build(nrows, ncols, 1024)(x)
